# baseline (speedup 1.0000x reference)
import functools

import jax
import jax.numpy as jnp
from jax import lax
from jax.experimental import pallas as pl
from jax.experimental.pallas import tpu as pltpu
from jax.experimental.pallas import tpu_sc as plsc

VOCAB_P1 = 1000001
EMBED_DIM = 32
BATCH = 16384

_info = plsc.get_sparse_core_info()
_NC, _NS = _info.num_cores, _info.num_subcores
_NW = _NC * _NS
_B_PER_W = BATCH // _NW


def _gather_body(idx_hbm, table_hbm, out_hbm, idx_v, rows_v, sem):
    wid = lax.axis_index("s") * _NC + lax.axis_index("c")
    base = wid * _B_PER_W
    pltpu.sync_copy(idx_hbm.at[pl.ds(base, _B_PER_W)], idx_v)
    pltpu.async_copy(table_hbm.at[idx_v], rows_v, sem).wait()
    pltpu.sync_copy(rows_v, out_hbm.at[pl.ds(base, _B_PER_W)])


@jax.jit
def kernel(item_id, id_table):
    mesh = plsc.VectorSubcoreMesh(core_axis_name="c", subcore_axis_name="s")
    gather = functools.partial(
        pl.kernel,
        mesh=mesh,
        out_type=jax.ShapeDtypeStruct((BATCH, EMBED_DIM), jnp.float32),
        scratch_types=[
            pltpu.VMEM((_B_PER_W,), jnp.int32),
            pltpu.VMEM((_B_PER_W, EMBED_DIM), jnp.float32),
            pltpu.SemaphoreType.DMA,
        ],
        compiler_params=pltpu.CompilerParams(use_tc_tiling_on_sc=False),
    )(_gather_body)
    return gather(item_id.astype(jnp.int32), id_table)


# speedup vs baseline: 3.9056x; 3.9056x over previous
import functools

import jax
import jax.numpy as jnp
from jax import lax
from jax.experimental import pallas as pl
from jax.experimental.pallas import tpu as pltpu
from jax.experimental.pallas import tpu_sc as plsc

VOCAB_P1 = 1000001
EMBED_DIM = 32
BATCH = 16384
_LANES = 128

_info = plsc.get_sparse_core_info()
_NC, _NS = _info.num_cores, _info.num_subcores
_NW = _NC * _NS
_B_PER_W = BATCH // _NW
_WAVE = 16
_N_WAVES = _B_PER_W // _WAVE


def _gather_body(idx_hbm, tab_hbm, out_hbm, idx_v, slab_v, out_v, sem):
    wid = lax.axis_index("s") * _NC + lax.axis_index("c")
    base = wid * _B_PER_W
    pltpu.sync_copy(idx_hbm.at[pl.ds(base, _B_PER_W)], idx_v)
    c_lo = lax.iota(jnp.int32, 16)
    c_hi = c_lo + 16

    def wave(g, carry):
        vec = idx_v[pl.ds(g * _WAVE, _WAVE)]
        copies = []
        for k in range(_WAVE):
            r = vec[k]
            blk = pl.multiple_of(r & ~(_LANES - 1), _LANES)
            copies.append(
                pltpu.async_copy(
                    tab_hbm.at[:, pl.ds(blk, _LANES)], slab_v.at[k], sem
                )
            )
        lane = vec & (_LANES - 1)
        for k in range(_WAVE):
            copies[k].wait()
            l_vec = jnp.full((16,), lane[k], dtype=jnp.int32)
            lo = plsc.load_gather(slab_v.at[k], [c_lo, l_vec])
            hi = plsc.load_gather(slab_v.at[k], [c_hi, l_vec])
            j_vec = jnp.full((16,), g * _WAVE + k, dtype=jnp.int32)
            plsc.store_scatter(out_v, [c_lo, j_vec], lo)
            plsc.store_scatter(out_v, [c_hi, j_vec], hi)
        return carry

    lax.fori_loop(0, _N_WAVES, wave, None)
    pltpu.sync_copy(out_v, out_hbm.at[:, pl.ds(base, _B_PER_W)])


@jax.jit
def kernel(item_id, id_table):
    mesh = plsc.VectorSubcoreMesh(core_axis_name="c", subcore_axis_name="s")
    gather = functools.partial(
        pl.kernel,
        mesh=mesh,
        out_type=jax.ShapeDtypeStruct((EMBED_DIM, BATCH), jnp.float32),
        scratch_types=[
            pltpu.VMEM((_B_PER_W,), jnp.int32),
            pltpu.VMEM((_WAVE, EMBED_DIM, _LANES), jnp.float32),
            pltpu.VMEM((EMBED_DIM, _B_PER_W), jnp.float32),
            pltpu.SemaphoreType.DMA,
        ],
        compiler_params=pltpu.CompilerParams(needs_layout_passes=False),
    )(_gather_body)
    out_t = gather(item_id.astype(jnp.int32), id_table.T)
    return out_t.T
